# Initial kernel scaffold; baseline (speedup 1.0000x reference)
import functools

import jax
import jax.numpy as jnp
from jax import lax
from jax.experimental import pallas as pl
from jax.experimental.pallas import tpu as pltpu
from jax.experimental.pallas import tpu_sc as plsc

B, N, D = 16, 2048, 64
K = 16
F1 = 128
PB = 256
PE = 256
NBK = N // PB
NBE = N // PE


def _bnsum_kernel(x_ref, s1_ref, s2_ref):
    b = pl.program_id(0)
    xb = x_ref[0]
    s1 = jnp.sum(xb, axis=0)
    s2 = jnp.sum(xb * xb, axis=0)

    @pl.when(b == 0)
    def _():
        s1_ref[0] = s1
        s2_ref[0] = s2

    @pl.when(b != 0)
    def _():
        s1_ref[0] = s1_ref[0] + s1
        s2_ref[0] = s2_ref[0] + s2


def _knn_kernel(x_ref, scale_ref, idx_ref):
    b = pl.program_id(0)
    j = pl.program_id(1)
    sc = scale_ref[0]
    xs = x_ref[0] * sc[None, :]
    sq = jnp.sum(xs * xs, axis=1)
    xi = xs[pl.ds(j * PB, PB), :]
    sqi = jnp.sum(xi * xi, axis=1)
    inner = lax.dot_general(xi, xs, (((1,), (1,)), ((), ())),
                            preferred_element_type=jnp.float32)
    d = sqi[:, None] - 2.0 * inner + sq[None, :]
    iota = lax.broadcasted_iota(jnp.int32, (PB, N), 1)
    base = b * N
    for kk in range(K):
        m = jnp.min(d, axis=1, keepdims=True)
        sel = d <= m
        idxv = jnp.min(jnp.where(sel, iota, N), axis=1)
        idx_ref[0, kk, :] = idxv + base
        d = jnp.where(sel, jnp.float32(jnp.inf), d)


def _sc_gather(x2d, idx2d):
    info = plsc.get_sparse_core_info()
    nc, ns = info.num_cores, info.num_subcores
    nw = nc * ns
    rows = idx2d.shape[0]
    rpw = rows // nw
    ch = 8
    nch = rpw // ch
    mesh = plsc.VectorSubcoreMesh(core_axis_name="c", subcore_axis_name="s")

    @functools.partial(
        pl.kernel, mesh=mesh,
        out_type=jax.ShapeDtypeStruct((rows, 128, D), jnp.float32),
        scratch_types=[
            pltpu.VMEM((ch, 128), jnp.int32),
            pltpu.VMEM((ch, 128, D), jnp.float32),
            pltpu.SemaphoreType.DMA,
        ],
    )
    def gk(table_hbm, idx_hbm, out_hbm, idx_v, rows_v, sem):
        wid = lax.axis_index("s") * nc + lax.axis_index("c")
        base = wid * rpw

        def body(c, carry):
            r0 = base + c * ch
            pltpu.sync_copy(idx_hbm.at[pl.ds(r0, ch)], idx_v)
            descs = [pltpu.async_copy(table_hbm.at[idx_v.at[jj]], rows_v.at[jj], sem)
                     for jj in range(ch)]
            for dsc in descs:
                dsc.wait()
            pltpu.sync_copy(rows_v, out_hbm.at[pl.ds(r0, ch)])
            return carry

        lax.fori_loop(0, nch, body, 0)

    return gk(x2d, idx2d)


def _edge_kernel(x_ref, g_ref, scale_ref, shift_ref, w1s_ref, b1_ref, w1b_ref,
                 w2_ref, b2_ref, w3_ref, b3_ref, vo_ref, pm_ref):
    j = pl.program_id(1)
    sc = scale_ref[0][None, :]
    sh = shift_ref[0][None, :]
    xi = x_ref[0] * sc + sh
    nb = g_ref[0].reshape(K * PE, D) * sc + sh
    ci = lax.dot_general(xi, w1s_ref[...], (((1,), (0,)), ((), ())),
                         preferred_element_type=jnp.float32) + b1_ref[0][None, :]
    y = lax.dot_general(nb, w1b_ref[...], (((1,), (0,)), ((), ())),
                        preferred_element_type=jnp.float32)
    h1 = jax.nn.relu(ci[None, :, :] - y.reshape(K, PE, F1))
    h2 = jax.nn.relu(
        lax.dot_general(h1.reshape(K * PE, F1), w2_ref[...], (((1,), (0,)), ((), ())),
                        preferred_element_type=jnp.float32) + b2_ref[0][None, :])
    h3 = jax.nn.relu(
        lax.dot_general(h2, w3_ref[...], (((1,), (0,)), ((), ())),
                        preferred_element_type=jnp.float32) + b3_ref[0][None, :])
    vo = jnp.max(h3.reshape(K, PE, D), axis=0)
    vo_ref[0] = vo
    pm = jnp.max(vo, axis=0)

    @pl.when(j == 0)
    def _():
        pm_ref[0, 0] = pm

    @pl.when(j != 0)
    def _():
        pm_ref[0, 0] = jnp.maximum(pm_ref[0, 0], pm)


def _final_kernel(vo_ref, pm_ref, wga_ref, wgb_ref, bg_ref, out_ref):
    g = pm_ref[0, 0]
    gc = lax.dot_general(g[None, :], wgb_ref[...], (((1,), (0,)), ((), ())),
                         preferred_element_type=jnp.float32)
    o = lax.dot_general(vo_ref[0], wga_ref[...], (((1,), (0,)), ((), ())),
                        preferred_element_type=jnp.float32)
    out_ref[0] = jax.nn.relu(o + gc + bg_ref[0][None, :])


def kernel(input_space, bn_gamma, bn_beta, W1, b1, W2, b2, W3, b3, Wg, bg):
    x = input_space
    f32 = jnp.float32

    s1, s2 = pl.pallas_call(
        _bnsum_kernel,
        grid=(B,),
        in_specs=[pl.BlockSpec((1, N, D), lambda b: (b, 0, 0))],
        out_specs=[pl.BlockSpec((1, D), lambda b: (0, 0))] * 2,
        out_shape=[jax.ShapeDtypeStruct((1, D), f32)] * 2,
    )(x)
    cnt = float(B * N)
    mean = s1 / cnt
    var = s2 / cnt - mean * mean
    rstd = lax.rsqrt(var + 1e-5)
    scale = bn_gamma[None, :] * rstd
    shift = bn_beta[None, :] - mean * scale

    idxg = pl.pallas_call(
        _knn_kernel,
        grid=(B, NBK),
        in_specs=[
            pl.BlockSpec((1, N, D), lambda b, j: (b, 0, 0)),
            pl.BlockSpec((1, D), lambda b, j: (0, 0)),
        ],
        out_specs=pl.BlockSpec((1, K, PB), lambda b, j: (b, 0, j)),
        out_shape=jax.ShapeDtypeStruct((B, K, N), jnp.int32),
    )(x, scale)

    x2d = x.reshape(B * N, D)
    idx2d = idxg.reshape((B * K * N) // 128, 128)
    gath = _sc_gather(x2d, idx2d).reshape(B, K, N, D)

    w1sT = (W1[:, :D] + W1[:, D:]).T
    w1bT = W1[:, D:].T
    w2T = W2.T
    w3T = W3.T
    wgaT = Wg[:, :D].T
    wgbT = Wg[:, D:].T

    vo, pm = pl.pallas_call(
        _edge_kernel,
        grid=(B, NBE),
        in_specs=[
            pl.BlockSpec((1, PE, D), lambda b, j: (b, j, 0)),
            pl.BlockSpec((1, K, PE, D), lambda b, j: (b, 0, j, 0)),
            pl.BlockSpec((1, D), lambda b, j: (0, 0)),
            pl.BlockSpec((1, D), lambda b, j: (0, 0)),
            pl.BlockSpec((D, F1), lambda b, j: (0, 0)),
            pl.BlockSpec((1, F1), lambda b, j: (0, 0)),
            pl.BlockSpec((D, F1), lambda b, j: (0, 0)),
            pl.BlockSpec((F1, D), lambda b, j: (0, 0)),
            pl.BlockSpec((1, D), lambda b, j: (0, 0)),
            pl.BlockSpec((D, D), lambda b, j: (0, 0)),
            pl.BlockSpec((1, D), lambda b, j: (0, 0)),
        ],
        out_specs=[
            pl.BlockSpec((1, PE, D), lambda b, j: (b, j, 0)),
            pl.BlockSpec((1, 1, D), lambda b, j: (b, 0, 0)),
        ],
        out_shape=[
            jax.ShapeDtypeStruct((B, N, D), f32),
            jax.ShapeDtypeStruct((B, 1, D), f32),
        ],
    )(x, gath, scale, shift, w1sT, b1[None, :], w1bT, w2T, b2[None, :], w3T,
      b3[None, :])

    out = pl.pallas_call(
        _final_kernel,
        grid=(B, NBE),
        in_specs=[
            pl.BlockSpec((1, PE, D), lambda b, j: (b, j, 0)),
            pl.BlockSpec((1, 1, D), lambda b, j: (b, 0, 0)),
            pl.BlockSpec((D, D), lambda b, j: (0, 0)),
            pl.BlockSpec((D, D), lambda b, j: (0, 0)),
            pl.BlockSpec((1, D), lambda b, j: (0, 0)),
        ],
        out_specs=pl.BlockSpec((1, PE, D), lambda b, j: (b, j, 0)),
        out_shape=jax.ShapeDtypeStruct((B, N, D), f32),
    )(vo, pm, wgaT, wgbT, bg[None, :])
    return out


# speedup vs baseline: 10.0183x; 10.0183x over previous
import functools

import jax
import jax.numpy as jnp
from jax import lax
from jax.experimental import pallas as pl
from jax.experimental.pallas import tpu as pltpu
from jax.experimental.pallas import tpu_sc as plsc

B, N, D = 16, 2048, 64
K = 16
F1 = 128
PB = 256
PE = 256
NBK = N // PB
NBE = N // PE


def _sum_kernel(x_ref, s_ref):
    b = pl.program_id(0)
    s = jnp.sum(x_ref[0], axis=0)

    @pl.when(b == 0)
    def _():
        s_ref[0] = s

    @pl.when(b != 0)
    def _():
        s_ref[0] = s_ref[0] + s


def _var_kernel(x_ref, mean_ref, s_ref):
    b = pl.program_id(0)
    c = x_ref[0] - mean_ref[0][None, :]
    s = jnp.sum(c * c, axis=0)

    @pl.when(b == 0)
    def _():
        s_ref[0] = s

    @pl.when(b != 0)
    def _():
        s_ref[0] = s_ref[0] + s


def _knn_kernel(x_ref, mean_ref, rstd_ref, gam_ref, bet_ref, idx_ref, xnp_ref):
    b = pl.program_id(0)
    j = pl.program_id(1)
    mean = mean_ref[0][None, :]
    rstd = rstd_ref[0][None, :]
    gam = gam_ref[0][None, :]
    bet = bet_ref[0][None, :]
    xn = ((x_ref[0] - mean) * rstd) * gam + bet
    sq = jnp.sum(xn * xn, axis=1)
    xi = ((x_ref[0, pl.ds(j * PB, PB), :] - mean) * rstd) * gam + bet
    sqi = jnp.sum(xi * xi, axis=1)
    inner = lax.dot_general(xi, xn, (((1,), (1,)), ((), ())))
    d = sqi[:, None] - 2.0 * inner + sq[None, :]

    @pl.when(j == 0)
    def _():
        xnp_ref[0] = jnp.concatenate([xn, jnp.zeros_like(xn)], axis=-1)

    iota = lax.broadcasted_iota(jnp.int32, (PB, N), 1)
    base = b * N
    for kk in range(K):
        m = jnp.min(d, axis=1, keepdims=True)
        sel = d <= m
        idxv = jnp.min(jnp.where(sel, iota, N), axis=1)
        idx_ref[0, kk, :] = idxv + base
        d = jnp.where(iota == idxv[:, None], jnp.float32(jnp.inf), d)


def _sc_gather(tab2d, idx1d):
    info = plsc.get_sparse_core_info()
    nc, ns = info.num_cores, info.num_subcores
    nw = nc * ns
    w = tab2d.shape[1]
    nf = idx1d.shape[0]
    fpw = nf // nw
    ch = 128
    nch = fpw // ch
    mesh = plsc.VectorSubcoreMesh(core_axis_name="c", subcore_axis_name="s")

    @functools.partial(
        pl.kernel, mesh=mesh,
        out_type=jax.ShapeDtypeStruct((nf, w), jnp.float32),
        scratch_types=[
            pltpu.VMEM((ch,), jnp.int32),
            pltpu.VMEM((ch, w), jnp.float32),
            pltpu.SemaphoreType.DMA,
        ],
    )
    def gk(table_hbm, idx_hbm, out_hbm, idx_v, rows_v, sem):
        wid = lax.axis_index("s") * nc + lax.axis_index("c")
        base = wid * fpw

        def body(c, carry):
            off = base + c * ch
            pltpu.sync_copy(idx_hbm.at[pl.ds(off, ch)], idx_v)
            pltpu.async_copy(table_hbm.at[idx_v], rows_v, sem).wait()
            pltpu.sync_copy(rows_v, out_hbm.at[pl.ds(off, ch)])
            return carry

        lax.fori_loop(0, nch, body, 0)

    return gk(tab2d, idx1d)


def _edge_kernel(xnp_ref, g_ref, w1_ref, b1_ref, w2_ref, b2_ref, w3_ref, b3_ref,
                 vo_ref, pm_ref):
    j = pl.program_id(1)
    xi = xnp_ref[0][:, :D]
    nj = g_ref[0][:, :, :D]
    xib = jnp.broadcast_to(xi[None], (K, PE, D))
    edge = jnp.concatenate([xib, xib - nj], axis=-1)
    h1 = jax.nn.relu(
        lax.dot_general(edge.reshape(K * PE, F1), w1_ref[...],
                        (((1,), (0,)), ((), ()))) + b1_ref[0][None, :])
    h2 = jax.nn.relu(
        lax.dot_general(h1, w2_ref[...], (((1,), (0,)), ((), ()))) + b2_ref[0][None, :])
    h3 = jax.nn.relu(
        lax.dot_general(h2, w3_ref[...], (((1,), (0,)), ((), ()))) + b3_ref[0][None, :])
    vo = jnp.max(h3.reshape(K, PE, D), axis=0)
    vo_ref[0] = vo
    pm = jnp.max(vo, axis=0)

    @pl.when(j == 0)
    def _():
        pm_ref[0, 0] = pm

    @pl.when(j != 0)
    def _():
        pm_ref[0, 0] = jnp.maximum(pm_ref[0, 0], pm)


def _final_kernel(vo_ref, pm_ref, wg_ref, bg_ref, out_ref):
    g = pm_ref[0, 0]
    cat = jnp.concatenate(
        [vo_ref[0], jnp.broadcast_to(g[None, :], (PE, D))], axis=-1)
    o = lax.dot_general(cat, wg_ref[...], (((1,), (0,)), ((), ())))
    out_ref[0] = jax.nn.relu(o + bg_ref[0][None, :])


def kernel(input_space, bn_gamma, bn_beta, W1, b1, W2, b2, W3, b3, Wg, bg):
    x = input_space
    f32 = jnp.float32

    s1 = pl.pallas_call(
        _sum_kernel,
        grid=(B,),
        in_specs=[pl.BlockSpec((1, N, D), lambda b: (b, 0, 0))],
        out_specs=pl.BlockSpec((1, D), lambda b: (0, 0)),
        out_shape=jax.ShapeDtypeStruct((1, D), f32),
    )(x)
    cnt = float(B * N)
    mean = s1 / cnt
    s2 = pl.pallas_call(
        _var_kernel,
        grid=(B,),
        in_specs=[pl.BlockSpec((1, N, D), lambda b: (b, 0, 0)),
                  pl.BlockSpec((1, D), lambda b: (0, 0))],
        out_specs=pl.BlockSpec((1, D), lambda b: (0, 0)),
        out_shape=jax.ShapeDtypeStruct((1, D), f32),
    )(x, mean)
    var = s2 / cnt
    rstd = 1.0 / jnp.sqrt(var + 1e-5)

    idxg, xnp = pl.pallas_call(
        _knn_kernel,
        grid=(B, NBK),
        in_specs=[
            pl.BlockSpec((1, N, D), lambda b, j: (b, 0, 0)),
            pl.BlockSpec((1, D), lambda b, j: (0, 0)),
            pl.BlockSpec((1, D), lambda b, j: (0, 0)),
            pl.BlockSpec((1, D), lambda b, j: (0, 0)),
            pl.BlockSpec((1, D), lambda b, j: (0, 0)),
        ],
        out_specs=[
            pl.BlockSpec((1, K, PB), lambda b, j: (b, 0, j)),
            pl.BlockSpec((1, N, 2 * D), lambda b, j: (b, 0, 0)),
        ],
        out_shape=[
            jax.ShapeDtypeStruct((B, K, N), jnp.int32),
            jax.ShapeDtypeStruct((B, N, 2 * D), f32),
        ],
    )(x, mean, rstd, bn_gamma[None, :], bn_beta[None, :])

    tab2d = xnp.reshape(B * N, 2 * D)
    idx1d = idxg.reshape(B * K * N)
    gath = _sc_gather(tab2d, idx1d).reshape(B, K, N, 2 * D)

    w1T = W1.T
    w2T = W2.T
    w3T = W3.T
    wgT = Wg.T

    vo, pm = pl.pallas_call(
        _edge_kernel,
        grid=(B, NBE),
        in_specs=[
            pl.BlockSpec((1, PE, 2 * D), lambda b, j: (b, j, 0)),
            pl.BlockSpec((1, K, PE, 2 * D), lambda b, j: (b, 0, j, 0)),
            pl.BlockSpec((F1, F1), lambda b, j: (0, 0)),
            pl.BlockSpec((1, F1), lambda b, j: (0, 0)),
            pl.BlockSpec((F1, D), lambda b, j: (0, 0)),
            pl.BlockSpec((1, D), lambda b, j: (0, 0)),
            pl.BlockSpec((D, D), lambda b, j: (0, 0)),
            pl.BlockSpec((1, D), lambda b, j: (0, 0)),
        ],
        out_specs=[
            pl.BlockSpec((1, PE, D), lambda b, j: (b, j, 0)),
            pl.BlockSpec((1, 1, D), lambda b, j: (b, 0, 0)),
        ],
        out_shape=[
            jax.ShapeDtypeStruct((B, N, D), f32),
            jax.ShapeDtypeStruct((B, 1, D), f32),
        ],
    )(xnp, gath, w1T, b1[None, :], w2T, b2[None, :], w3T, b3[None, :])

    out = pl.pallas_call(
        _final_kernel,
        grid=(B, NBE),
        in_specs=[
            pl.BlockSpec((1, PE, D), lambda b, j: (b, j, 0)),
            pl.BlockSpec((1, 1, D), lambda b, j: (b, 0, 0)),
            pl.BlockSpec((F1, D), lambda b, j: (0, 0)),
            pl.BlockSpec((1, D), lambda b, j: (0, 0)),
        ],
        out_specs=pl.BlockSpec((1, PE, D), lambda b, j: (b, j, 0)),
        out_shape=jax.ShapeDtypeStruct((B, N, D), f32),
    )(vo, pm, wgT, bg[None, :])
    return out


# speedup vs baseline: 11.5339x; 1.1513x over previous
import functools

import jax
import jax.numpy as jnp
from jax import lax
from jax.experimental import pallas as pl
from jax.experimental.pallas import tpu as pltpu
from jax.experimental.pallas import tpu_sc as plsc

B, N, D = 16, 2048, 64
K = 16
F1 = 128
PB = 256
PE = 256
NBK = N // PB
NBE = N // PE


def _sum_kernel(x_ref, s_ref):
    b = pl.program_id(0)
    s = jnp.sum(x_ref[0], axis=0)

    @pl.when(b == 0)
    def _():
        s_ref[0] = s

    @pl.when(b != 0)
    def _():
        s_ref[0] = s_ref[0] + s


def _var_kernel(x_ref, mean_ref, s_ref):
    b = pl.program_id(0)
    c = x_ref[0] - mean_ref[0][None, :]
    s = jnp.sum(c * c, axis=0)

    @pl.when(b == 0)
    def _():
        s_ref[0] = s

    @pl.when(b != 0)
    def _():
        s_ref[0] = s_ref[0] + s


def _knn_kernel(x_ref, mean_ref, rstd_ref, gam_ref, bet_ref, idx_ref, xnp_ref):
    b = pl.program_id(0)
    j = pl.program_id(1)
    mean = mean_ref[0][None, :]
    rstd = rstd_ref[0][None, :]
    gam = gam_ref[0][None, :]
    bet = bet_ref[0][None, :]
    xn = ((x_ref[0] - mean) * rstd) * gam + bet
    sq = jnp.sum(xn * xn, axis=1)
    xi = ((x_ref[0, pl.ds(j * PB, PB), :] - mean) * rstd) * gam + bet
    sqi = jnp.sum(xi * xi, axis=1)
    inner = lax.dot_general(xi, xn, (((1,), (1,)), ((), ())))
    d = sqi[:, None] - 2.0 * inner + sq[None, :]

    @pl.when(j == 0)
    def _():
        xnp_ref[0] = jnp.concatenate([xn, jnp.zeros_like(xn)], axis=-1)

    iota = lax.broadcasted_iota(jnp.int32, (PB, N), 1)
    base = b * N
    for kk in range(K):
        m = jnp.min(d, axis=1, keepdims=True)
        sel = d <= m
        idxv = jnp.min(jnp.where(sel, iota, N), axis=1)
        idx_ref[0, kk, :] = idxv + base
        d = jnp.where(iota == idxv[:, None], jnp.float32(jnp.inf), d)


def _sc_gather(tab2d, idx1d):
    info = plsc.get_sparse_core_info()
    nc, ns = info.num_cores, info.num_subcores
    nw = nc * ns
    w = tab2d.shape[1]
    nf = idx1d.shape[0]
    fpw = nf // nw
    ch = 128
    nch = fpw // ch
    mesh = plsc.VectorSubcoreMesh(core_axis_name="c", subcore_axis_name="s")

    @functools.partial(
        pl.kernel, mesh=mesh,
        out_type=jax.ShapeDtypeStruct((nf, w), jnp.float32),
        scratch_types=[
            pltpu.VMEM((ch,), jnp.int32),
            pltpu.VMEM((ch, w), jnp.float32),
            pltpu.SemaphoreType.DMA,
        ],
    )
    def gk(table_hbm, idx_hbm, out_hbm, idx_v, rows_v, sem):
        wid = lax.axis_index("s") * nc + lax.axis_index("c")
        base = wid * fpw

        def body(c, carry):
            off = base + c * ch
            pltpu.sync_copy(idx_hbm.at[pl.ds(off, ch)], idx_v)
            pltpu.async_copy(table_hbm.at[idx_v], rows_v, sem).wait()
            pltpu.sync_copy(rows_v, out_hbm.at[pl.ds(off, ch)])
            return carry

        lax.fori_loop(0, nch, body, 0)

    return gk(tab2d, idx1d)


def _edge_kernel(xnp_ref, g_ref, w1_ref, b1_ref, w2_ref, b2_ref, w3_ref, b3_ref,
                 vo_ref, pm_ref):
    j = pl.program_id(1)
    xi = xnp_ref[0][:, :D]
    nj = g_ref[0][:, :, :D]
    xib = jnp.broadcast_to(xi[None], (K, PE, D))
    edge = jnp.concatenate([xib, xib - nj], axis=-1)
    h1 = jax.nn.relu(
        lax.dot_general(edge.reshape(K * PE, F1), w1_ref[...],
                        (((1,), (0,)), ((), ()))) + b1_ref[0][None, :])
    h2 = jax.nn.relu(
        lax.dot_general(h1, w2_ref[...], (((1,), (0,)), ((), ()))) + b2_ref[0][None, :])
    h3 = jax.nn.relu(
        lax.dot_general(h2, w3_ref[...], (((1,), (0,)), ((), ()))) + b3_ref[0][None, :])
    vo = jnp.max(h3.reshape(K, PE, D), axis=0)
    vo_ref[0] = vo
    pm = jnp.max(vo, axis=0)

    @pl.when(j == 0)
    def _():
        pm_ref[0, 0] = pm

    @pl.when(j != 0)
    def _():
        pm_ref[0, 0] = jnp.maximum(pm_ref[0, 0], pm)


def _final_kernel(vo_ref, pm_ref, wg_ref, bg_ref, out_ref):
    g = pm_ref[0, 0]
    cat = jnp.concatenate(
        [vo_ref[0], jnp.broadcast_to(g[None, :], (PE, D))], axis=-1)
    o = lax.dot_general(cat, wg_ref[...], (((1,), (0,)), ((), ())))
    out_ref[0] = jax.nn.relu(o + bg_ref[0][None, :])


def kernel(input_space, bn_gamma, bn_beta, W1, b1, W2, b2, W3, b3, Wg, bg):
    x = input_space
    f32 = jnp.float32

    s1 = pl.pallas_call(
        _sum_kernel,
        grid=(B,),
        in_specs=[pl.BlockSpec((1, N, D), lambda b: (b, 0, 0))],
        out_specs=pl.BlockSpec((1, D), lambda b: (0, 0)),
        out_shape=jax.ShapeDtypeStruct((1, D), f32),
    )(x)
    cnt = float(B * N)
    mean = s1 / cnt
    s2 = pl.pallas_call(
        _var_kernel,
        grid=(B,),
        in_specs=[pl.BlockSpec((1, N, D), lambda b: (b, 0, 0)),
                  pl.BlockSpec((1, D), lambda b: (0, 0))],
        out_specs=pl.BlockSpec((1, D), lambda b: (0, 0)),
        out_shape=jax.ShapeDtypeStruct((1, D), f32),
    )(x, mean)
    var = s2 / cnt
    rstd = 1.0 / jnp.sqrt(var + 1e-5)

    w1T = W1.T
    w2T = W2.T
    w3T = W3.T
    wgT = Wg.T

    gb = 4
    ng = B // gb
    gam2 = bn_gamma[None, :]
    bet2 = bn_beta[None, :]

    knn_outs = []
    for g in range(ng):
        xg = lax.slice_in_dim(x, g * gb, (g + 1) * gb, axis=0)
        idxg, xnp = pl.pallas_call(
            _knn_kernel,
            grid=(gb, NBK),
            in_specs=[
                pl.BlockSpec((1, N, D), lambda b, j: (b, 0, 0)),
                pl.BlockSpec((1, D), lambda b, j: (0, 0)),
                pl.BlockSpec((1, D), lambda b, j: (0, 0)),
                pl.BlockSpec((1, D), lambda b, j: (0, 0)),
                pl.BlockSpec((1, D), lambda b, j: (0, 0)),
            ],
            out_specs=[
                pl.BlockSpec((1, K, PB), lambda b, j: (b, 0, j)),
                pl.BlockSpec((1, N, 2 * D), lambda b, j: (b, 0, 0)),
            ],
            out_shape=[
                jax.ShapeDtypeStruct((gb, K, N), jnp.int32),
                jax.ShapeDtypeStruct((gb, N, 2 * D), f32),
            ],
        )(xg, mean, rstd, gam2, bet2)
        knn_outs.append((idxg, xnp))

    gath_outs = []
    for g in range(ng):
        idxg, xnp = knn_outs[g]
        tab2d = xnp.reshape(gb * N, 2 * D)
        idx1d = idxg.reshape(gb * K * N)
        gath_outs.append(_sc_gather(tab2d, idx1d).reshape(gb, K, N, 2 * D))

    outs = []
    for g in range(ng):
        idxg, xnp = knn_outs[g]
        gath = gath_outs[g]
        vo, pm = pl.pallas_call(
            _edge_kernel,
            grid=(gb, NBE),
            in_specs=[
                pl.BlockSpec((1, PE, 2 * D), lambda b, j: (b, j, 0)),
                pl.BlockSpec((1, K, PE, 2 * D), lambda b, j: (b, 0, j, 0)),
                pl.BlockSpec((F1, F1), lambda b, j: (0, 0)),
                pl.BlockSpec((1, F1), lambda b, j: (0, 0)),
                pl.BlockSpec((F1, D), lambda b, j: (0, 0)),
                pl.BlockSpec((1, D), lambda b, j: (0, 0)),
                pl.BlockSpec((D, D), lambda b, j: (0, 0)),
                pl.BlockSpec((1, D), lambda b, j: (0, 0)),
            ],
            out_specs=[
                pl.BlockSpec((1, PE, D), lambda b, j: (b, j, 0)),
                pl.BlockSpec((1, 1, D), lambda b, j: (b, 0, 0)),
            ],
            out_shape=[
                jax.ShapeDtypeStruct((gb, N, D), f32),
                jax.ShapeDtypeStruct((gb, 1, D), f32),
            ],
        )(xnp, gath, w1T, b1[None, :], w2T, b2[None, :], w3T, b3[None, :])

        outs.append(pl.pallas_call(
            _final_kernel,
            grid=(gb, NBE),
            in_specs=[
                pl.BlockSpec((1, PE, D), lambda b, j: (b, j, 0)),
                pl.BlockSpec((1, 1, D), lambda b, j: (b, 0, 0)),
                pl.BlockSpec((F1, D), lambda b, j: (0, 0)),
                pl.BlockSpec((1, D), lambda b, j: (0, 0)),
            ],
            out_specs=pl.BlockSpec((1, PE, D), lambda b, j: (b, j, 0)),
            out_shape=jax.ShapeDtypeStruct((gb, N, D), f32),
        )(vo, pm, wgT, bg[None, :]))
    return jnp.concatenate(outs, axis=0)


# speedup vs baseline: 14.8978x; 1.2917x over previous
import functools

import jax
import jax.numpy as jnp
from jax import lax
from jax.experimental import pallas as pl
from jax.experimental.pallas import tpu as pltpu
from jax.experimental.pallas import tpu_sc as plsc

B, N, D = 16, 2048, 64
K = 16
F1 = 128
PB = 256
PE = 256
NBK = N // PB
NBE = N // PE


def _sum_kernel(x_ref, s_ref):
    b = pl.program_id(0)
    s = jnp.sum(x_ref[0], axis=0)

    @pl.when(b == 0)
    def _():
        s_ref[0] = s

    @pl.when(b != 0)
    def _():
        s_ref[0] = s_ref[0] + s


def _var_kernel(x_ref, mean_ref, s_ref):
    b = pl.program_id(0)
    c = x_ref[0] - mean_ref[0][None, :]
    s = jnp.sum(c * c, axis=0)

    @pl.when(b == 0)
    def _():
        s_ref[0] = s

    @pl.when(b != 0)
    def _():
        s_ref[0] = s_ref[0] + s


def _knn_kernel(x_ref, mean_ref, rstd_ref, gam_ref, bet_ref, idx_ref, xnp_ref):
    b = pl.program_id(0)
    j = pl.program_id(1)
    mean = mean_ref[0][None, :]
    rstd = rstd_ref[0][None, :]
    gam = gam_ref[0][None, :]
    bet = bet_ref[0][None, :]
    xn = ((x_ref[0] - mean) * rstd) * gam + bet
    sq = jnp.sum(xn * xn, axis=1)
    xi = ((x_ref[0, pl.ds(j * PB, PB), :] - mean) * rstd) * gam + bet
    sqi = jnp.sum(xi * xi, axis=1)
    inner = lax.dot_general(xi, xn, (((1,), (1,)), ((), ())))
    d = sqi[:, None] - 2.0 * inner + sq[None, :]

    @pl.when(j == 0)
    def _():
        xnp_ref[0] = jnp.concatenate([xn, jnp.zeros_like(xn)], axis=-1)

    iotaf = lax.broadcasted_iota(jnp.int32, (PB, N), 1).astype(jnp.float32)
    base = b * N
    for kk in range(K):
        m = jnp.min(d, axis=1, keepdims=True)
        encf = jnp.where(d <= m, iotaf, jnp.float32(N))
        idxf = jnp.min(encf, axis=1, keepdims=True)
        idx_ref[0, :, kk] = idxf[:, 0].astype(jnp.int32) + base
        d = jnp.where(encf == idxf, jnp.float32(jnp.inf), d)


def _sc_gather(tab2d, idx1d):
    info = plsc.get_sparse_core_info()
    nc, ns = info.num_cores, info.num_subcores
    nw = nc * ns
    w = tab2d.shape[1]
    nf = idx1d.shape[0]
    fpw = nf // nw
    ch = 128
    nch = fpw // ch
    mesh = plsc.VectorSubcoreMesh(core_axis_name="c", subcore_axis_name="s")

    @functools.partial(
        pl.kernel, mesh=mesh,
        out_type=jax.ShapeDtypeStruct((nf, w), jnp.float32),
        scratch_types=[
            pltpu.VMEM((ch,), jnp.int32),
            pltpu.VMEM((ch, w), jnp.float32),
            pltpu.SemaphoreType.DMA,
        ],
    )
    def gk(table_hbm, idx_hbm, out_hbm, idx_v, rows_v, sem):
        wid = lax.axis_index("s") * nc + lax.axis_index("c")
        base = wid * fpw

        def body(c, carry):
            off = base + c * ch
            pltpu.sync_copy(idx_hbm.at[pl.ds(off, ch)], idx_v)
            pltpu.async_copy(table_hbm.at[idx_v], rows_v, sem).wait()
            pltpu.sync_copy(rows_v, out_hbm.at[pl.ds(off, ch)])
            return carry

        lax.fori_loop(0, nch, body, 0)

    return gk(tab2d, idx1d)


def _edge_kernel(xnp_ref, g_ref, w1_ref, b1_ref, w2_ref, b2_ref, w3_ref, b3_ref,
                 vo_ref, pm_ref):
    j = pl.program_id(1)
    xi = xnp_ref[0][:, :D]
    nj = g_ref[0][:, :, :D]
    xib = jnp.broadcast_to(xi[:, None, :], (PE, K, D))
    edge = jnp.concatenate([xib, xib - nj], axis=-1)
    h1 = jax.nn.relu(
        lax.dot_general(edge.reshape(PE * K, F1), w1_ref[...],
                        (((1,), (0,)), ((), ()))) + b1_ref[0][None, :])
    h2 = jax.nn.relu(
        lax.dot_general(h1, w2_ref[...], (((1,), (0,)), ((), ()))) + b2_ref[0][None, :])
    h3 = jax.nn.relu(
        lax.dot_general(h2, w3_ref[...], (((1,), (0,)), ((), ()))) + b3_ref[0][None, :])
    vo = jnp.max(h3.reshape(PE, K, D), axis=1)
    vo_ref[0] = vo
    pm = jnp.max(vo, axis=0)

    @pl.when(j == 0)
    def _():
        pm_ref[0, 0] = pm

    @pl.when(j != 0)
    def _():
        pm_ref[0, 0] = jnp.maximum(pm_ref[0, 0], pm)


def _final_kernel(vo_ref, pm_ref, wg_ref, bg_ref, out_ref):
    g = pm_ref[0, 0]
    cat = jnp.concatenate(
        [vo_ref[0], jnp.broadcast_to(g[None, :], (PE, D))], axis=-1)
    o = lax.dot_general(cat, wg_ref[...], (((1,), (0,)), ((), ())))
    out_ref[0] = jax.nn.relu(o + bg_ref[0][None, :])


def kernel(input_space, bn_gamma, bn_beta, W1, b1, W2, b2, W3, b3, Wg, bg):
    x = input_space
    f32 = jnp.float32

    s1 = pl.pallas_call(
        _sum_kernel,
        grid=(B,),
        in_specs=[pl.BlockSpec((1, N, D), lambda b: (b, 0, 0))],
        out_specs=pl.BlockSpec((1, D), lambda b: (0, 0)),
        out_shape=jax.ShapeDtypeStruct((1, D), f32),
    )(x)
    cnt = float(B * N)
    mean = s1 / cnt
    s2 = pl.pallas_call(
        _var_kernel,
        grid=(B,),
        in_specs=[pl.BlockSpec((1, N, D), lambda b: (b, 0, 0)),
                  pl.BlockSpec((1, D), lambda b: (0, 0))],
        out_specs=pl.BlockSpec((1, D), lambda b: (0, 0)),
        out_shape=jax.ShapeDtypeStruct((1, D), f32),
    )(x, mean)
    var = s2 / cnt
    rstd = 1.0 / jnp.sqrt(var + 1e-5)

    w1T = W1.T
    w2T = W2.T
    w3T = W3.T
    wgT = Wg.T

    gb = 4
    ng = B // gb
    gam2 = bn_gamma[None, :]
    bet2 = bn_beta[None, :]

    knn_outs = []
    for g in range(ng):
        xg = lax.slice_in_dim(x, g * gb, (g + 1) * gb, axis=0)
        idxg, xnp = pl.pallas_call(
            _knn_kernel,
            grid=(gb, NBK),
            in_specs=[
                pl.BlockSpec((1, N, D), lambda b, j: (b, 0, 0)),
                pl.BlockSpec((1, D), lambda b, j: (0, 0)),
                pl.BlockSpec((1, D), lambda b, j: (0, 0)),
                pl.BlockSpec((1, D), lambda b, j: (0, 0)),
                pl.BlockSpec((1, D), lambda b, j: (0, 0)),
            ],
            out_specs=[
                pl.BlockSpec((1, PB, K), lambda b, j: (b, j, 0)),
                pl.BlockSpec((1, N, 2 * D), lambda b, j: (b, 0, 0)),
            ],
            out_shape=[
                jax.ShapeDtypeStruct((gb, N, K), jnp.int32),
                jax.ShapeDtypeStruct((gb, N, 2 * D), f32),
            ],
        )(xg, mean, rstd, gam2, bet2)
        knn_outs.append((idxg, xnp))

    gath_outs = []
    for g in range(ng):
        idxg, xnp = knn_outs[g]
        tab2d = xnp.reshape(gb * N, 2 * D)
        idx1d = idxg.reshape(gb * K * N)
        gath_outs.append(_sc_gather(tab2d, idx1d).reshape(gb, N, K, 2 * D))

    outs = []
    for g in range(ng):
        idxg, xnp = knn_outs[g]
        gath = gath_outs[g]
        vo, pm = pl.pallas_call(
            _edge_kernel,
            grid=(gb, NBE),
            in_specs=[
                pl.BlockSpec((1, PE, 2 * D), lambda b, j: (b, j, 0)),
                pl.BlockSpec((1, PE, K, 2 * D), lambda b, j: (b, j, 0, 0)),
                pl.BlockSpec((F1, F1), lambda b, j: (0, 0)),
                pl.BlockSpec((1, F1), lambda b, j: (0, 0)),
                pl.BlockSpec((F1, D), lambda b, j: (0, 0)),
                pl.BlockSpec((1, D), lambda b, j: (0, 0)),
                pl.BlockSpec((D, D), lambda b, j: (0, 0)),
                pl.BlockSpec((1, D), lambda b, j: (0, 0)),
            ],
            out_specs=[
                pl.BlockSpec((1, PE, D), lambda b, j: (b, j, 0)),
                pl.BlockSpec((1, 1, D), lambda b, j: (b, 0, 0)),
            ],
            out_shape=[
                jax.ShapeDtypeStruct((gb, N, D), f32),
                jax.ShapeDtypeStruct((gb, 1, D), f32),
            ],
        )(xnp, gath, w1T, b1[None, :], w2T, b2[None, :], w3T, b3[None, :])

        outs.append(pl.pallas_call(
            _final_kernel,
            grid=(gb, NBE),
            in_specs=[
                pl.BlockSpec((1, PE, D), lambda b, j: (b, j, 0)),
                pl.BlockSpec((1, 1, D), lambda b, j: (b, 0, 0)),
                pl.BlockSpec((F1, D), lambda b, j: (0, 0)),
                pl.BlockSpec((1, D), lambda b, j: (0, 0)),
            ],
            out_specs=pl.BlockSpec((1, PE, D), lambda b, j: (b, j, 0)),
            out_shape=jax.ShapeDtypeStruct((gb, N, D), f32),
        )(vo, pm, wgT, bg[None, :]))
    return jnp.concatenate(outs, axis=0)
